# baseline (speedup 1.0000x reference)
import functools

import jax
import jax.numpy as jnp
from jax import lax
from jax.experimental import pallas as pl
from jax.experimental.pallas import tpu as pltpu
from jax.experimental.pallas import tpu_sc as plsc

_INF8 = 255
_H = 16
_B = 16
_N = 512
_NP1 = _N + 1
_L = 16
_NC = 2
_NS = 16
_NW = _NC * _NS
_ROWS_PER_W = _B * _N // _NW
_R = 8
_CHUNKS = _ROWS_PER_W // _R
_PAIRS = _CHUNKS // 2
_VPR = _N // _L
_NG = 4
_HG = _H // _NG
_BPAIRS_PER_W = _B * _H // _NW


def _sc_embed_body(ab_hbm, tbank_hbm, vspl_hbm, out_hbm,
                   tbl_v, vspl_v, idx_v, buf_v, bot_v,
                   semo0, semo1, semi0, semi1, semb):
    wid = lax.axis_index("s") * _NC + lax.axis_index("c")
    pltpu.sync_copy(tbank_hbm, tbl_v)
    pltpu.sync_copy(vspl_hbm, vspl_v)

    semo = (semo0, semo1)
    semi = (semi0, semi1)
    odd = lax.rem(wid, 2)
    b = wid // 2
    row0 = wid * _ROWS_PER_W
    lane = lax.iota(jnp.int32, _L)
    lane8 = lane * 8
    last_lane = lane == (_L - 1)

    for t in range(_BPAIRS_PER_W):
        vlo = vspl_v[t, pl.ds(0, _L)]
        vhi = vspl_v[t + _H // 2, pl.ds(0, _L)]
        vh = jnp.where(odd == 0, vlo, vhi)
        for j in range(_VPR):
            bot_v[t, 0, pl.ds(j * _L, _L)] = vh
        bot_v[t, 0, pl.ds(_N - _L + 1, _L)] = vh

    def idx_issue(chunk, slot):
        gr = row0 + chunk * _R
        pltpu.async_copy(
            ab_hbm.at[pl.ds(gr * _N, _R * _N)], idx_v.at[slot], semi[slot])

    def idx_wait(slot):
        pltpu.make_async_copy(
            ab_hbm.at[pl.ds(0, _R * _N)], idx_v.at[slot], semi[slot]).wait()

    def out_refs(g, h, i0):
        return (buf_v.at[g % 2, h],
                out_hbm.at[b, g * _HG + h, pl.ds(i0, _R), :])

    def gather_section(g, slot, i0):
        q = g % 2

        @plsc.parallel_loop(0, _R * _VPR, unroll=4)
        def _(j):
            r = lax.shift_right_logical(j, 5)
            k = lax.bitwise_and(j, _VPR - 1)
            iv = idx_v[slot, pl.ds(j * _L, _L)]
            ivb = iv * (_L * 8) + lane8
            for h in range(_HG):
                off = (g // 2) * 32768 + (g % 2) * _HG + h
                vals = plsc.load_gather(tbl_v, [ivb + off])
                buf_v[q, h, r, pl.ds(k * _L, _L)] = vals
        for h in range(_HG):
            vh = vspl_v[g * _HG + h, pl.ds(0, _L)]
            for r in range(_R):
                seg = buf_v[q, h, r, pl.ds(_N - _L + 1, _L)]
                buf_v[q, h, r, pl.ds(_N - _L + 1, _L)] = (
                    jnp.where(last_lane, vh, seg))
        for h in range(_HG):
            src, dst = out_refs(g, h, i0)
            pltpu.async_copy(src, dst, semo[q])

    def drain_section(g, i0):
        for h in range(_HG):
            src, dst = out_refs(g, h, i0)
            pltpu.make_async_copy(src, dst, semo[g % 2]).wait()

    idx_issue(0, 0)

    def pair_body(p, carry):
        c0 = 2 * p
        i00 = odd * _ROWS_PER_W + c0 * _R
        i01 = i00 + _R
        idx_issue(c0 + 1, 1)
        idx_wait(0)
        for g in range(_NG):
            if g < 2:
                @pl.when(p >= 1)
                def _(g=g):
                    drain_section(g + _NG - 2, i00)
            else:
                drain_section(g - 2, i00)
            gather_section(g, 0, i00)
        idx_issue(jnp.where(p < _PAIRS - 1, c0 + 2, 0), 0)
        idx_wait(1)
        for g in range(_NG):
            if g < 2:
                drain_section(g + _NG - 2, i00)
            else:
                drain_section(g - 2, i01)
            gather_section(g, 1, i01)
        return carry

    lax.fori_loop(0, _PAIRS, pair_body, 0)

    i_last = odd * _ROWS_PER_W + (_CHUNKS - 1) * _R
    for g in range(_NG - 2, _NG):
        drain_section(g, i_last)
    idx_wait(0)

    hbase = odd * (_H // 2)
    cps = [
        pltpu.async_copy(
            bot_v.at[t], out_hbm.at[b, hbase + t, pl.ds(_N, 1), :], semb)
        for t in range(_BPAIRS_PER_W)
    ]
    for cp in cps:
        cp.wait()


@functools.lru_cache(maxsize=1)
def _sc_embed():
    return pl.kernel(
        _sc_embed_body,
        out_type=jax.ShapeDtypeStruct((_B, _H, _NP1, _NP1), jnp.float32),
        mesh=plsc.VectorSubcoreMesh(core_axis_name="c", subcore_axis_name="s",
                                    num_cores=_NC, num_subcores=_NS),
        compiler_params=pltpu.CompilerParams(needs_layout_passes=False),
        scratch_types=[
            pltpu.VMEM((_H * 256 * _L,), jnp.float32),
            pltpu.VMEM((_H, _L), jnp.float32),
            pltpu.VMEM((2, _R * _N), jnp.int32),
            pltpu.VMEM((2, _HG, _R, _NP1), jnp.float32),
            pltpu.VMEM((_BPAIRS_PER_W, 1, _NP1), jnp.float32),
            pltpu.SemaphoreType.DMA,
            pltpu.SemaphoreType.DMA,
            pltpu.SemaphoreType.DMA,
            pltpu.SemaphoreType.DMA,
            pltpu.SemaphoreType.DMA,
        ],
    )


def kernel(attn_bias, linear_bias_w, virtual_bias_w):
    ab_flat = attn_bias.reshape(_B * _N * _N)
    tmod = linear_bias_w.at[_INF8].set(-jnp.inf)
    tgh = tmod.T.reshape(2, 8, 256).transpose(0, 2, 1)
    tbank = jnp.broadcast_to(tgh[:, :, None, :], (2, 256, _L, 8))
    vspl = jnp.broadcast_to(virtual_bias_w.reshape(_H, 1), (_H, _L))
    return _sc_embed()(ab_flat, tbank.reshape(-1), vspl)


# speedup vs baseline: 1.2220x; 1.2220x over previous
import functools

import jax
import jax.numpy as jnp
from jax import lax
from jax.experimental import pallas as pl
from jax.experimental.pallas import tpu as pltpu
from jax.experimental.pallas import tpu_sc as plsc

_INF8 = 255
_H = 16
_B = 16
_N = 512
_NP1 = _N + 1
_L = 16
_NC = 2
_NS = 16
_NW = _NC * _NS
_ROWS_PER_W = _B * _N // _NW
_R = 8
_CHUNKS = _ROWS_PER_W // _R
_PAIRS = _CHUNKS // 2
_VPR = _N // _L
_NG = 4
_HG = _H // _NG
_BPAIRS_PER_W = _B * _H // _NW


def _sc_embed_body(ab_hbm, tbank_hbm, vspl_hbm, out_hbm,
                   tbl_v, vspl_v, idx_v, buf_v, bot_v,
                   semo0, semo1, semi0, semi1, semb):
    wid = lax.axis_index("s") * _NC + lax.axis_index("c")
    pltpu.sync_copy(tbank_hbm, tbl_v)
    pltpu.sync_copy(vspl_hbm, vspl_v)

    semo = (semo0, semo1)
    semi = (semi0, semi1)
    odd = lax.rem(wid, 2)
    b = wid // 2
    row0 = wid * _ROWS_PER_W
    lane = lax.iota(jnp.int32, _L)
    last_lane = lane == (_L - 1)

    for t in range(_BPAIRS_PER_W):
        vlo = vspl_v[t, pl.ds(0, _L)]
        vhi = vspl_v[t + _H // 2, pl.ds(0, _L)]
        vh = jnp.where(odd == 0, vlo, vhi)
        for j in range(_VPR):
            bot_v[t, 0, pl.ds(j * _L, _L)] = vh
        bot_v[t, 0, pl.ds(_N - _L + 1, _L)] = vh

    def idx_issue(chunk, slot):
        gr = row0 + chunk * _R
        pltpu.async_copy(
            ab_hbm.at[pl.ds(gr * _N, _R * _N)], idx_v.at[slot], semi[slot])

    def idx_wait(slot):
        pltpu.make_async_copy(
            ab_hbm.at[pl.ds(0, _R * _N)], idx_v.at[slot], semi[slot]).wait()

    def out_refs(g, h, i0):
        return (buf_v.at[g % 2, h],
                out_hbm.at[b, g * _HG + h, pl.ds(i0, _R), :])

    def gather_section(g, slot, i0):
        q = g % 2

        @plsc.parallel_loop(0, _R * _VPR, unroll=4)
        def _(j):
            r = lax.shift_right_logical(j, 5)
            k = lax.bitwise_and(j, _VPR - 1)
            iv = idx_v[slot, pl.ds(j * _L, _L)]
            ivb = iv * _L + lane
            for pr in range(_HG // 2):
                w = plsc.load_gather(tbl_v, [ivb + (g * 2 + pr) * 4096])
                lo = plsc.bitcast(jnp.left_shift(w, 16), jnp.float32)
                hi = plsc.bitcast(jnp.bitwise_and(w, -65536), jnp.float32)
                buf_v[q, 2 * pr, r, pl.ds(k * _L, _L)] = lo
                buf_v[q, 2 * pr + 1, r, pl.ds(k * _L, _L)] = hi
        for h in range(_HG):
            vh = vspl_v[g * _HG + h, pl.ds(0, _L)]
            for r in range(_R):
                seg = buf_v[q, h, r, pl.ds(_N - _L + 1, _L)]
                buf_v[q, h, r, pl.ds(_N - _L + 1, _L)] = (
                    jnp.where(last_lane, vh, seg))
        for h in range(_HG):
            src, dst = out_refs(g, h, i0)
            pltpu.async_copy(src, dst, semo[q])

    def drain_section(g, i0):
        for h in range(_HG):
            src, dst = out_refs(g, h, i0)
            pltpu.make_async_copy(src, dst, semo[g % 2]).wait()

    idx_issue(0, 0)

    def pair_body(p, carry):
        c0 = 2 * p
        i00 = odd * _ROWS_PER_W + c0 * _R
        i01 = i00 + _R
        idx_issue(c0 + 1, 1)
        idx_wait(0)
        for g in range(_NG):
            if g < 2:
                @pl.when(p >= 1)
                def _(g=g):
                    drain_section(g + _NG - 2, i00)
            else:
                drain_section(g - 2, i00)
            gather_section(g, 0, i00)
        idx_issue(jnp.where(p < _PAIRS - 1, c0 + 2, 0), 0)
        idx_wait(1)
        for g in range(_NG):
            if g < 2:
                drain_section(g + _NG - 2, i00)
            else:
                drain_section(g - 2, i01)
            gather_section(g, 1, i01)
        return carry

    lax.fori_loop(0, _PAIRS, pair_body, 0)

    i_last = odd * _ROWS_PER_W + (_CHUNKS - 1) * _R
    for g in range(_NG - 2, _NG):
        drain_section(g, i_last)
    idx_wait(0)

    hbase = odd * (_H // 2)
    cps = [
        pltpu.async_copy(
            bot_v.at[t], out_hbm.at[b, hbase + t, pl.ds(_N, 1), :], semb)
        for t in range(_BPAIRS_PER_W)
    ]
    for cp in cps:
        cp.wait()


@functools.lru_cache(maxsize=1)
def _sc_embed():
    return pl.kernel(
        _sc_embed_body,
        out_type=jax.ShapeDtypeStruct((_B, _H, _NP1, _NP1), jnp.float32),
        mesh=plsc.VectorSubcoreMesh(core_axis_name="c", subcore_axis_name="s",
                                    num_cores=_NC, num_subcores=_NS),
        compiler_params=pltpu.CompilerParams(needs_layout_passes=False),
        scratch_types=[
            pltpu.VMEM((_H // 2 * 256 * _L,), jnp.int32),
            pltpu.VMEM((_H, _L), jnp.float32),
            pltpu.VMEM((2, _R * _N), jnp.int32),
            pltpu.VMEM((2, _HG, _R, _NP1), jnp.float32),
            pltpu.VMEM((_BPAIRS_PER_W, 1, _NP1), jnp.float32),
            pltpu.SemaphoreType.DMA,
            pltpu.SemaphoreType.DMA,
            pltpu.SemaphoreType.DMA,
            pltpu.SemaphoreType.DMA,
            pltpu.SemaphoreType.DMA,
        ],
    )


def kernel(attn_bias, linear_bias_w, virtual_bias_w):
    ab_flat = attn_bias.reshape(_B * _N * _N)
    tmod = linear_bias_w.at[_INF8].set(-jnp.inf)
    bits = lax.bitcast_convert_type(
        tmod.astype(jnp.bfloat16), jnp.uint16).astype(jnp.uint32)
    packed = bits[:, 0::2] | (bits[:, 1::2] << 16)
    tbank = jnp.broadcast_to(packed.T[:, :, None], (_H // 2, 256, _L))
    tbank = lax.bitcast_convert_type(tbank, jnp.int32)
    vspl = jnp.broadcast_to(virtual_bias_w.reshape(_H, 1), (_H, _L))
    return _sc_embed()(ab_flat, tbank.reshape(-1), vspl)
